# Initial kernel scaffold; baseline (speedup 1.0000x reference)
import jax
import jax.numpy as jnp
from jax.experimental import pallas as pl


def kernel(x, selection_index, selection_probabilities, W_bank, b_bank):
    raise NotImplementedError("write your pallas kernel here")


# speedup vs baseline: 2.4260x; 2.4260x over previous
import functools

import jax
import jax.numpy as jnp
from jax.experimental import pallas as pl
from jax.experimental.pallas import tpu as pltpu

_B, _S, _IN_F, _OUT_F, _BANK, _K = 4, 2048, 1024, 1024, 16, 2
_S_TILE = 512
_O_TILE = 512


def _body(sel_ref, p_ref, x_ref, w0_ref, w1_ref, b0_ref, b1_ref, o_ref):
    b = pl.program_id(0)
    p0 = p_ref[b, 0]
    p1 = p_ref[b, 1]
    w = p0 * w0_ref[0] + p1 * w1_ref[0]
    acc = jnp.dot(x_ref[0], w, preferred_element_type=jnp.float32)
    bias = p0 * b0_ref[0] + p1 * b1_ref[0]
    o_ref[0] = acc + bias


def kernel(x, selection_index, selection_probabilities, W_bank, b_bank):
    sel = selection_index.astype(jnp.int32)
    p = selection_probabilities.astype(jnp.float32)
    b3 = b_bank.reshape(_BANK, 1, _OUT_F)
    grid = (_B, _OUT_F // _O_TILE, _S // _S_TILE)

    grid_spec = pltpu.PrefetchScalarGridSpec(
        num_scalar_prefetch=2,
        grid=grid,
        in_specs=[
            pl.BlockSpec((1, _S_TILE, _IN_F), lambda b, o, s, sel, p: (b, s, 0)),
            pl.BlockSpec((1, _IN_F, _O_TILE), lambda b, o, s, sel, p: (sel[b, 0], 0, o)),
            pl.BlockSpec((1, _IN_F, _O_TILE), lambda b, o, s, sel, p: (sel[b, 1], 0, o)),
            pl.BlockSpec((1, 1, _O_TILE), lambda b, o, s, sel, p: (sel[b, 0], 0, o)),
            pl.BlockSpec((1, 1, _O_TILE), lambda b, o, s, sel, p: (sel[b, 1], 0, o)),
        ],
        out_specs=pl.BlockSpec((1, _S_TILE, _O_TILE), lambda b, o, s, sel, p: (b, s, o)),
    )

    return pl.pallas_call(
        _body,
        grid_spec=grid_spec,
        out_shape=jax.ShapeDtypeStruct((_B, _S, _OUT_F), jnp.float32),
    )(sel, p, x, W_bank, W_bank, b3, b3)


# speedup vs baseline: 3.2208x; 1.3277x over previous
import functools

import jax
import jax.numpy as jnp
from jax.experimental import pallas as pl
from jax.experimental.pallas import tpu as pltpu

_B, _S, _IN_F, _OUT_F, _BANK, _K = 4, 2048, 1024, 1024, 16, 2
_S_TILE = 512
_O_TILE = 1024


def _body(sel_ref, p_ref, x_ref, w0_ref, w1_ref, b0_ref, b1_ref, o_ref):
    b = pl.program_id(0)
    p0 = p_ref[b, 0]
    p1 = p_ref[b, 1]
    w = p0 * w0_ref[0] + p1 * w1_ref[0]
    acc = jnp.dot(x_ref[0], w, preferred_element_type=jnp.float32)
    bias = p0 * b0_ref[0] + p1 * b1_ref[0]
    o_ref[0] = acc + bias


def kernel(x, selection_index, selection_probabilities, W_bank, b_bank):
    sel = selection_index.astype(jnp.int32)
    p = selection_probabilities.astype(jnp.float32)
    b3 = b_bank.reshape(_BANK, 1, _OUT_F)
    grid = (_B, _OUT_F // _O_TILE, _S // _S_TILE)

    grid_spec = pltpu.PrefetchScalarGridSpec(
        num_scalar_prefetch=2,
        grid=grid,
        in_specs=[
            pl.BlockSpec((1, _S_TILE, _IN_F), lambda b, o, s, sel, p: (b, s, 0)),
            pl.BlockSpec((1, _IN_F, _O_TILE), lambda b, o, s, sel, p: (sel[b, 0], 0, o)),
            pl.BlockSpec((1, _IN_F, _O_TILE), lambda b, o, s, sel, p: (sel[b, 1], 0, o)),
            pl.BlockSpec((1, 1, _O_TILE), lambda b, o, s, sel, p: (sel[b, 0], 0, o)),
            pl.BlockSpec((1, 1, _O_TILE), lambda b, o, s, sel, p: (sel[b, 1], 0, o)),
        ],
        out_specs=pl.BlockSpec((1, _S_TILE, _O_TILE), lambda b, o, s, sel, p: (b, s, o)),
    )

    return pl.pallas_call(
        _body,
        grid_spec=grid_spec,
        out_shape=jax.ShapeDtypeStruct((_B, _S, _OUT_F), jnp.float32),
    )(sel, p, x, W_bank, W_bank, b3, b3)


# speedup vs baseline: 3.4650x; 1.0758x over previous
import functools

import jax
import jax.numpy as jnp
from jax.experimental import pallas as pl
from jax.experimental.pallas import tpu as pltpu

_B, _S, _IN_F, _OUT_F, _BANK, _K = 4, 2048, 1024, 1024, 16, 2
_S_TILE = 1024
_O_TILE = 1024


def _body(sel_ref, p_ref, x_ref, w0_ref, w1_ref, b0_ref, b1_ref, o_ref):
    b = pl.program_id(0)
    p0 = p_ref[b, 0]
    p1 = p_ref[b, 1]
    w = p0 * w0_ref[0] + p1 * w1_ref[0]
    acc = jnp.dot(x_ref[0], w, preferred_element_type=jnp.float32)
    bias = p0 * b0_ref[0] + p1 * b1_ref[0]
    o_ref[0] = acc + bias


def kernel(x, selection_index, selection_probabilities, W_bank, b_bank):
    sel = selection_index.astype(jnp.int32)
    p = selection_probabilities.astype(jnp.float32)
    b3 = b_bank.reshape(_BANK, 1, _OUT_F)
    grid = (_B, _OUT_F // _O_TILE, _S // _S_TILE)

    grid_spec = pltpu.PrefetchScalarGridSpec(
        num_scalar_prefetch=2,
        grid=grid,
        in_specs=[
            pl.BlockSpec((1, _S_TILE, _IN_F), lambda b, o, s, sel, p: (b, s, 0)),
            pl.BlockSpec((1, _IN_F, _O_TILE), lambda b, o, s, sel, p: (sel[b, 0], 0, o)),
            pl.BlockSpec((1, _IN_F, _O_TILE), lambda b, o, s, sel, p: (sel[b, 1], 0, o)),
            pl.BlockSpec((1, 1, _O_TILE), lambda b, o, s, sel, p: (sel[b, 0], 0, o)),
            pl.BlockSpec((1, 1, _O_TILE), lambda b, o, s, sel, p: (sel[b, 1], 0, o)),
        ],
        out_specs=pl.BlockSpec((1, _S_TILE, _O_TILE), lambda b, o, s, sel, p: (b, s, o)),
    )

    return pl.pallas_call(
        _body,
        grid_spec=grid_spec,
        out_shape=jax.ShapeDtypeStruct((_B, _S, _OUT_F), jnp.float32),
    )(sel, p, x, W_bank, W_bank, b3, b3)


# speedup vs baseline: 3.9727x; 1.1465x over previous
import functools

import jax
import jax.numpy as jnp
from jax.experimental import pallas as pl
from jax.experimental.pallas import tpu as pltpu

_B, _S, _IN_F, _OUT_F, _BANK, _K = 4, 2048, 1024, 1024, 16, 2
_S_TILE = 2048
_O_TILE = 1024


def _body(sel_ref, p_ref, x_ref, w0_ref, w1_ref, b0_ref, b1_ref, o_ref):
    b = pl.program_id(0)
    p0 = p_ref[b, 0]
    p1 = p_ref[b, 1]
    w = p0 * w0_ref[0] + p1 * w1_ref[0]
    acc = jnp.dot(x_ref[0], w, preferred_element_type=jnp.float32)
    bias = p0 * b0_ref[0] + p1 * b1_ref[0]
    o_ref[0] = acc + bias


def kernel(x, selection_index, selection_probabilities, W_bank, b_bank):
    sel = selection_index.astype(jnp.int32)
    p = selection_probabilities.astype(jnp.float32)
    b3 = b_bank.reshape(_BANK, 1, _OUT_F)
    grid = (_B, _OUT_F // _O_TILE, _S // _S_TILE)

    grid_spec = pltpu.PrefetchScalarGridSpec(
        num_scalar_prefetch=2,
        grid=grid,
        in_specs=[
            pl.BlockSpec((1, _S_TILE, _IN_F), lambda b, o, s, sel, p: (b, s, 0)),
            pl.BlockSpec((1, _IN_F, _O_TILE), lambda b, o, s, sel, p: (sel[b, 0], 0, o)),
            pl.BlockSpec((1, _IN_F, _O_TILE), lambda b, o, s, sel, p: (sel[b, 1], 0, o)),
            pl.BlockSpec((1, 1, _O_TILE), lambda b, o, s, sel, p: (sel[b, 0], 0, o)),
            pl.BlockSpec((1, 1, _O_TILE), lambda b, o, s, sel, p: (sel[b, 1], 0, o)),
        ],
        out_specs=pl.BlockSpec((1, _S_TILE, _O_TILE), lambda b, o, s, sel, p: (b, s, o)),
    )

    return pl.pallas_call(
        _body,
        grid_spec=grid_spec,
        out_shape=jax.ShapeDtypeStruct((_B, _S, _OUT_F), jnp.float32),
    )(sel, p, x, W_bank, W_bank, b3, b3)


# speedup vs baseline: 3.9776x; 1.0012x over previous
import functools

import jax
import jax.numpy as jnp
from jax.experimental import pallas as pl
from jax.experimental.pallas import tpu as pltpu

_B, _S, _IN_F, _OUT_F, _BANK, _K = 4, 2048, 1024, 1024, 16, 2
_S_TILE = 2048
_O_TILE = 1024


def _body(sel_ref, p_ref, x_ref, w0_ref, w1_ref, b0_ref, b1_ref, o_ref):
    b = pl.program_id(0)
    p0 = p_ref[b, 0]
    p1 = p_ref[b, 1]
    w = p0 * w0_ref[0] + p1 * w1_ref[0]
    acc = jnp.dot(x_ref[0].astype(jnp.bfloat16), w.astype(jnp.bfloat16),
                  preferred_element_type=jnp.float32)
    bias = p0 * b0_ref[0] + p1 * b1_ref[0]
    o_ref[0] = acc + bias


def kernel(x, selection_index, selection_probabilities, W_bank, b_bank):
    sel = selection_index.astype(jnp.int32)
    p = selection_probabilities.astype(jnp.float32)
    b3 = b_bank.reshape(_BANK, 1, _OUT_F)
    grid = (_B, _OUT_F // _O_TILE, _S // _S_TILE)

    grid_spec = pltpu.PrefetchScalarGridSpec(
        num_scalar_prefetch=2,
        grid=grid,
        in_specs=[
            pl.BlockSpec((1, _S_TILE, _IN_F), lambda b, o, s, sel, p: (b, s, 0)),
            pl.BlockSpec((1, _IN_F, _O_TILE), lambda b, o, s, sel, p: (sel[b, 0], 0, o)),
            pl.BlockSpec((1, _IN_F, _O_TILE), lambda b, o, s, sel, p: (sel[b, 1], 0, o)),
            pl.BlockSpec((1, 1, _O_TILE), lambda b, o, s, sel, p: (sel[b, 0], 0, o)),
            pl.BlockSpec((1, 1, _O_TILE), lambda b, o, s, sel, p: (sel[b, 1], 0, o)),
        ],
        out_specs=pl.BlockSpec((1, _S_TILE, _O_TILE), lambda b, o, s, sel, p: (b, s, o)),
    )

    return pl.pallas_call(
        _body,
        grid_spec=grid_spec,
        out_shape=jax.ShapeDtypeStruct((_B, _S, _OUT_F), jnp.float32),
    )(sel, p, x, W_bank, W_bank, b3, b3)
